# baseline (speedup 1.0000x reference)
import jax
import jax.numpy as jnp
from jax.experimental import pallas as pl
from jax.experimental.pallas import tpu as pltpu

NUM_EXPERTS = 64
TOP_K = 8
EPS = 1e-06
BLOCK_M = 1024


def _router_body(x_ref, wt_ref, b_ref, o_ref):
    logits = jnp.dot(x_ref[...], wt_ref[...],
                     preferred_element_type=jnp.float32)
    lt = logits.T + b_ref[...]
    m = jnp.max(lt, axis=0, keepdims=True)
    e = jnp.exp(lt - m)
    z = jnp.sum(e, axis=0, keepdims=True)
    work = lt
    neg = jnp.float32(-jnp.inf)
    t = None
    for _ in range(TOP_K):
        t = jnp.max(work, axis=0, keepdims=True)
        work = jnp.where(work == t, neg, work)
    mask = (lt >= t).astype(jnp.float32)
    me = e * mask
    s = jnp.sum(me, axis=0, keepdims=True) + EPS * z
    o_ref[...] = (me / s).T


@jax.jit
def kernel(x, W, b):
    B, S, D = x.shape
    M = B * S
    x2 = x.reshape(M, D)
    wt = W.T
    b2 = b.reshape(NUM_EXPERTS, 1)
    grid = (M // BLOCK_M,)
    out = pl.pallas_call(
        _router_body,
        grid=grid,
        in_specs=[
            pl.BlockSpec((BLOCK_M, D), lambda i: (i, 0)),
            pl.BlockSpec((D, NUM_EXPERTS), lambda i: (0, 0)),
            pl.BlockSpec((NUM_EXPERTS, 1), lambda i: (0, 0)),
        ],
        out_specs=pl.BlockSpec((BLOCK_M, NUM_EXPERTS), lambda i: (i, 0)),
        out_shape=jax.ShapeDtypeStruct((M, NUM_EXPERTS), jnp.float32),
        compiler_params=pltpu.CompilerParams(
            dimension_semantics=("arbitrary",)),
    )(x2, wt, b2)
    return out.reshape(B, S, NUM_EXPERTS)


# speedup vs baseline: 1.0244x; 1.0244x over previous
import jax
import jax.numpy as jnp
from jax.experimental import pallas as pl
from jax.experimental.pallas import tpu as pltpu

NUM_EXPERTS = 64
TOP_K = 8
EPS = 1e-06
BLOCK_M = 512


def _router_body(x_ref, wt_ref, b_ref, o_ref):
    logits = jnp.dot(x_ref[...], wt_ref[...],
                     preferred_element_type=jnp.float32)
    lt = logits.T + b_ref[...]
    m = jnp.max(lt, axis=0, keepdims=True)
    e = jnp.exp(lt - m)
    z = jnp.sum(e, axis=0, keepdims=True)
    work = lt
    neg = jnp.float32(-jnp.inf)
    t = None
    for _ in range(TOP_K):
        t = jnp.max(work, axis=0, keepdims=True)
        work = jnp.where(work == t, neg, work)
    mask = (lt >= t).astype(jnp.float32)
    me = e * mask
    s = jnp.sum(me, axis=0, keepdims=True) + EPS * z
    o_ref[...] = me / s


@jax.jit
def kernel(x, W, b):
    B, S, D = x.shape
    M = B * S
    x2 = x.reshape(M, D)
    wt = W.T
    b2 = b.reshape(NUM_EXPERTS, 1)
    grid = (M // BLOCK_M,)
    out = pl.pallas_call(
        _router_body,
        grid=grid,
        in_specs=[
            pl.BlockSpec((BLOCK_M, D), lambda i: (i, 0)),
            pl.BlockSpec((D, NUM_EXPERTS), lambda i: (0, 0)),
            pl.BlockSpec((NUM_EXPERTS, 1), lambda i: (0, 0)),
        ],
        out_specs=pl.BlockSpec((NUM_EXPERTS, BLOCK_M), lambda i: (0, i)),
        out_shape=jax.ShapeDtypeStruct((NUM_EXPERTS, M), jnp.float32),
        compiler_params=pltpu.CompilerParams(
            dimension_semantics=("arbitrary",)),
    )(x2, wt, b2)
    return out.T.reshape(B, S, NUM_EXPERTS)


# speedup vs baseline: 1.0931x; 1.0670x over previous
import jax
import jax.numpy as jnp
from jax.experimental import pallas as pl
from jax.experimental.pallas import tpu as pltpu

NUM_EXPERTS = 64
TOP_K = 8
EPS = 1e-06
BLOCK_M = 1024


def _router_body(x_ref, w_ref, b_ref, o_ref):
    lt = jax.lax.dot_general(
        w_ref[...], x_ref[...],
        dimension_numbers=(((1,), (1,)), ((), ())),
        preferred_element_type=jnp.float32) + b_ref[...]
    m = jnp.max(lt, axis=0, keepdims=True)
    e = jnp.exp(lt - m)
    z = jnp.sum(e, axis=0, keepdims=True)
    work = lt
    neg = jnp.float32(-jnp.inf)
    t = None
    for _ in range(TOP_K):
        t = jnp.max(work, axis=0, keepdims=True)
        work = jnp.where(work == t, neg, work)
    mask = (lt >= t).astype(jnp.float32)
    me = e * mask
    s = jnp.sum(me, axis=0, keepdims=True) + EPS * z
    o_ref[...] = me / s


@jax.jit
def kernel(x, W, b):
    B, S, D = x.shape
    M = B * S
    x2 = x.reshape(M, D)
    b2 = b.reshape(NUM_EXPERTS, 1)
    grid = (M // BLOCK_M,)
    out = pl.pallas_call(
        _router_body,
        grid=grid,
        in_specs=[
            pl.BlockSpec((BLOCK_M, D), lambda i: (i, 0)),
            pl.BlockSpec((NUM_EXPERTS, D), lambda i: (0, 0)),
            pl.BlockSpec((NUM_EXPERTS, 1), lambda i: (0, 0)),
        ],
        out_specs=pl.BlockSpec((NUM_EXPERTS, BLOCK_M), lambda i: (0, i)),
        out_shape=jax.ShapeDtypeStruct((NUM_EXPERTS, M), jnp.float32),
        compiler_params=pltpu.CompilerParams(
            dimension_semantics=("arbitrary",)),
    )(x2, W, b2)
    return out.T.reshape(B, S, NUM_EXPERTS)


# speedup vs baseline: 1.0934x; 1.0003x over previous
import jax
import jax.numpy as jnp
from jax.experimental import pallas as pl
from jax.experimental.pallas import tpu as pltpu

NUM_EXPERTS = 64
TOP_K = 8
EPS = 1e-06
BLOCK_M = 512


def _route_block(x_blk, w, b):
    lt = jax.lax.dot_general(
        w, x_blk,
        dimension_numbers=(((1,), (1,)), ((), ())),
        preferred_element_type=jnp.float32) + b
    m = jnp.max(lt, axis=0, keepdims=True)
    e = jnp.exp(lt - m)
    z = jnp.sum(e, axis=0, keepdims=True)
    work = lt
    neg = jnp.float32(-jnp.inf)
    t = None
    for _ in range(TOP_K):
        t = jnp.max(work, axis=0, keepdims=True)
        work = jnp.where(work == t, neg, work)
    mask = (lt >= t).astype(jnp.float32)
    me = e * mask
    s = jnp.sum(me, axis=0, keepdims=True) + EPS * z
    return me / s


def _router_body(xa_ref, xb_ref, w_ref, b_ref, o_ref):
    w = w_ref[...]
    b = b_ref[...]
    o_ref[:, :BLOCK_M] = _route_block(xa_ref[...], w, b)
    o_ref[:, BLOCK_M:] = _route_block(xb_ref[...], w, b)


@jax.jit
def kernel(x, W, b):
    B, S, D = x.shape
    M = B * S
    x2 = x.reshape(M, D)
    b2 = b.reshape(NUM_EXPERTS, 1)
    grid = (M // (2 * BLOCK_M),)
    out = pl.pallas_call(
        _router_body,
        grid=grid,
        in_specs=[
            pl.BlockSpec((BLOCK_M, D), lambda i: (2 * i, 0)),
            pl.BlockSpec((BLOCK_M, D), lambda i: (2 * i + 1, 0)),
            pl.BlockSpec((NUM_EXPERTS, D), lambda i: (0, 0)),
            pl.BlockSpec((NUM_EXPERTS, 1), lambda i: (0, 0)),
        ],
        out_specs=pl.BlockSpec((NUM_EXPERTS, 2 * BLOCK_M), lambda i: (0, i)),
        out_shape=jax.ShapeDtypeStruct((NUM_EXPERTS, M), jnp.float32),
        compiler_params=pltpu.CompilerParams(
            dimension_semantics=("arbitrary",)),
    )(x2, x2, W, b2)
    return out.T.reshape(B, S, NUM_EXPERTS)
